# baseline (speedup 1.0000x reference)
import functools

import jax
import jax.numpy as jnp
from jax import lax
from jax.experimental import pallas as pl
from jax.experimental.pallas import tpu as pltpu, tpu_sc as plsc

N = 10000
E = 320000
C = 2000
B = 8
H = 128
NCLS = 10

NP = 10240
CPX = 2048
ET = 10240
EP = 32 * ET
EK = 128
NCH = ET // EK
RB = 512

_mesh = plsc.VectorSubcoreMesh(core_axis_name="c", subcore_axis_name="s")


def _z16f():
    return jnp.full((16,), 0.0, dtype=jnp.float32)


def _z16i():
    return jnp.full((16,), 0, dtype=jnp.int32)


def _sc_pre(ntc, batch, srcp, dstp, wp):
    NT = NP // 16

    @functools.partial(
        pl.kernel,
        out_type=[
            jax.ShapeDtypeStruct((EP,), jnp.int32),
            jax.ShapeDtypeStruct((EP,), jnp.int32),
            jax.ShapeDtypeStruct((2, 1, NP), jnp.float32),
            jax.ShapeDtypeStruct((2, 1, CPX), jnp.float32),
            jax.ShapeDtypeStruct((2, 1, CPX), jnp.int32),
        ],
        mesh=_mesh,
        scratch_types=[
            pltpu.VMEM((EK,), jnp.int32),
            pltpu.VMEM((EK,), jnp.int32),
            pltpu.VMEM((EK,), jnp.float32),
            pltpu.VMEM((EK,), jnp.int32),
            pltpu.VMEM((EK,), jnp.int32),
            pltpu.VMEM((NT,), jnp.int32),
            pltpu.VMEM((NT,), jnp.int32),
            pltpu.VMEM((CPX,), jnp.int32),
            pltpu.VMEM((640,), jnp.float32),
            pltpu.MemorySpace.VMEM_SHARED((NP,), jnp.float32),
            pltpu.MemorySpace.VMEM_SHARED((CPX,), jnp.float32),
            pltpu.MemorySpace.VMEM_SHARED((16, 1, CPX), jnp.int32),
            pltpu.SemaphoreType.DMA,
            pltpu.SemaphoreType.DMA,
        ],
    )
    def k(ntc_h, bat_h, src_h, dst_h, w_h, psrc_h, pdst_h, deg1_h, deg2_h, cb_h,
          src_v, dst_v, w_v, ps_v, pd_v, ntc_v, bat_v, cb_v, zv,
          deg1_sh, deg2_sh, cb_st, sem, sem2):
        cid = lax.axis_index("c")
        sid = lax.axis_index("s")
        wid = sid * 2 + cid

        def zb(j, carry):
            zv[pl.ds(j * 16, 16)] = _z16f()
            return carry
        lax.fori_loop(0, 40, zb, 0)
        pltpu.sync_copy(zv, deg1_sh.at[pl.ds(sid * 640, 640)])
        pltpu.sync_copy(zv.at[pl.ds(0, 128)], deg2_sh.at[pl.ds(sid * 128, 128)])

        def zcb(j, carry):
            cb_v[pl.ds(j * 16, 16)] = _z16i()
            return carry
        lax.fori_loop(0, CPX // 16, zcb, 0)

        nbase = wid * (NP // 32)
        pltpu.sync_copy(ntc_h.at[pl.ds(nbase, 320)], ntc_v.at[pl.ds(0, 320)])
        pltpu.sync_copy(bat_h.at[pl.ds(nbase, 320)], bat_v.at[pl.ds(0, 320)])
        lanes = lax.iota(jnp.int32, 16)

        def nb(g, carry):
            c16 = ntc_v[pl.ds(g * 16, 16)]
            b16 = bat_v[pl.ds(g * 16, 16)]
            for kk in range(16):
                c = c16[kk]
                b = b16[kk]
                ca = (c >> 4) << 4
                lane = c - ca
                cur = cb_v[pl.ds(ca, 16)]
                cb_v[pl.ds(ca, 16)] = jnp.where(
                    lanes == lane, jnp.maximum(cur, b), cur)
            return carry
        lax.fori_loop(0, 20, nb, 0)
        pltpu.sync_copy(cb_v, cb_st.at[sid, 0])
        plsc.subcore_barrier()

        ebase = wid * ET

        def eb(ch, carry):
            off = ebase + ch * EK
            pltpu.sync_copy(src_h.at[pl.ds(off, EK)], src_v)
            pltpu.sync_copy(dst_h.at[pl.ds(off, EK)], dst_v)
            pltpu.sync_copy(w_h.at[pl.ds(off, EK)], w_v)
            cp1 = pltpu.async_copy(ntc_h.at[src_v], ps_v, sem)
            cp2 = pltpu.async_copy(ntc_h.at[dst_v], pd_v, sem2)
            cp1.wait()
            cp2.wait()
            pltpu.sync_copy(ps_v, psrc_h.at[pl.ds(off, EK)])
            pltpu.sync_copy(pd_v, pdst_h.at[pl.ds(off, EK)])
            pltpu.sync_copy(w_v, deg1_sh.at[dst_v], add=True)
            pltpu.sync_copy(w_v, deg2_sh.at[pd_v], add=True)
            return carry
        lax.fori_loop(0, NCH, eb, 0)
        plsc.subcore_barrier()

        cslice = sid * 128
        pltpu.sync_copy(cb_st.at[0, 0, pl.ds(cslice, 128)],
                        cb_v.at[pl.ds(0, 128)])
        for t in range(1, 16):
            pltpu.sync_copy(
                cb_st.at[t, 0, pl.ds(cslice, 128)], cb_v.at[pl.ds(128, 128)])
            for j in range(0, 128, 16):
                cb_v[pl.ds(j, 16)] = jnp.maximum(
                    cb_v[pl.ds(j, 16)], cb_v[pl.ds(128 + j, 16)])
        pltpu.sync_copy(cb_v.at[pl.ds(0, 128)],
                        cb_h.at[cid, 0, pl.ds(cslice, 128)])
        pltpu.sync_copy(deg1_sh.at[pl.ds(sid * 640, 640)],
                        deg1_h.at[cid, 0, pl.ds(sid * 640, 640)])
        pltpu.sync_copy(deg2_sh.at[pl.ds(sid * 128, 128)],
                        deg2_h.at[cid, 0, pl.ds(sid * 128, 128)])

    return k(ntc, batch, srcp, dstp, wp)


def _sc_agg(table, src3, dst3, w3, npad):
    RT = npad // 16
    GC = 20

    @functools.partial(
        pl.kernel,
        out_type=jax.ShapeDtypeStruct((2, npad, H), jnp.float32),
        mesh=_mesh,
        scratch_types=[
            pltpu.VMEM((GC, 1, EK), jnp.int32),
            pltpu.VMEM((GC, 1, EK), jnp.int32),
            pltpu.VMEM((GC, 1, EK), jnp.float32),
            pltpu.VMEM((2, EK, H), jnp.float32),
            pltpu.VMEM((16, H), jnp.float32),
            pltpu.MemorySpace.VMEM_SHARED((npad, H), jnp.float32),
            [pltpu.SemaphoreType.DMA] * 2,
        ],
    )
    def k(tab_h, src_h, dst_h, w_h, out_h, srcg, dstg, wg, rows_v, zbuf,
          acc_sh, gsem):
        cid = lax.axis_index("c")
        sid = lax.axis_index("s")
        wid = sid * 2 + cid
        for r in range(16):
            for j in range(0, H, 16):
                zbuf[r, pl.ds(j, 16)] = _z16f()

        def zb(z, carry):
            pltpu.sync_copy(zbuf, acc_sh.at[pl.ds(sid * RT + z * 16, 16)])
            return carry
        lax.fori_loop(0, RT // 16, zb, 0)
        plsc.subcore_barrier()

        cbase = wid * NCH

        def gather(ch, slot):
            pltpu.async_copy(tab_h.at[srcg.at[ch, 0]], rows_v.at[slot],
                             gsem[slot])

        def consume(ch, slot):
            pltpu.make_async_copy(
                tab_h.at[srcg.at[ch, 0]], rows_v.at[slot],
                gsem[slot]).wait()

            def sc(g, carry2):
                w16 = wg[ch, 0, pl.ds(g * 16, 16)]
                for kk in range(16):
                    e = g * 16 + kk
                    wk = w16[kk]
                    for j in range(0, H, 16):
                        rows_v[slot, e, pl.ds(j, 16)] = (
                            rows_v[slot, e, pl.ds(j, 16)] * wk)
                return carry2
            lax.fori_loop(0, EK // 16, sc, 0, unroll=2)
            pltpu.sync_copy(rows_v.at[slot], acc_sh.at[dstg.at[ch, 0]],
                            add=True)

        def grp(g, carry):
            rowbase = cbase + g * GC
            pltpu.sync_copy(src_h.at[pl.ds(rowbase, GC)], srcg)
            pltpu.sync_copy(dst_h.at[pl.ds(rowbase, GC)], dstg)
            pltpu.sync_copy(w_h.at[pl.ds(rowbase, GC)], wg)
            gather(0, 0)

            def body(p, carry2):
                for s in range(2):
                    ch = 2 * p + s

                    @pl.when(ch + 1 < GC)
                    def _():
                        gather(ch + 1, 1 - s)
                    consume(ch, s)
                return carry2
            lax.fori_loop(0, GC // 2, body, 0)
            return carry
        lax.fori_loop(0, NCH // GC, grp, 0)
        plsc.subcore_barrier()
        pltpu.sync_copy(acc_sh.at[pl.ds(sid * RT, RT)],
                        out_h.at[cid, pl.ds(sid * RT, RT)])

    return k(table, src3, dst3, w3)


def _sc_pool1(h_fc, ntc, batch):
    RP = NP // 4
    HCH = 128
    CL = CPX * 16
    BL = B * 16

    @functools.partial(
        pl.kernel,
        out_type=[
            jax.ShapeDtypeStruct((2, 8, 1, CL), jnp.float32),
            jax.ShapeDtypeStruct((2, 8, 1, CL), jnp.float32),
            jax.ShapeDtypeStruct((2, 8, 1, BL), jnp.float32),
            jax.ShapeDtypeStruct((2, 8, 1, BL), jnp.float32),
        ],
        mesh=_mesh,
        scratch_types=[
            pltpu.VMEM((HCH, 16), jnp.float32),
            pltpu.VMEM((RP,), jnp.int32),
            pltpu.VMEM((RP,), jnp.int32),
            pltpu.VMEM((CL,), jnp.float32),
            pltpu.VMEM((CL,), jnp.float32),
            pltpu.VMEM((CL // 4,), jnp.float32),
            pltpu.VMEM((BL,), jnp.float32),
            pltpu.VMEM((BL,), jnp.float32),
            pltpu.MemorySpace.VMEM_SHARED((8, 1, CL), jnp.float32),
            pltpu.MemorySpace.VMEM_SHARED((8, 1, BL), jnp.float32),
            pltpu.MemorySpace.VMEM_SHARED((8, 1, BL), jnp.float32),
        ],
    )
    def k(h_h, ntc_h, bat_h, xadd_h, xmax_h, bsum_h, bmax_h,
          hbuf, ntc_v, bat_v, accs, accm, cmb, bs, bm, st_s, st_bs,
          st_bm):
        cid = lax.axis_index("c")
        sid = lax.axis_index("s")
        nr = sid // 8
        fc = sid % 8
        row0 = cid * (2 * RP) + nr * RP
        pltpu.sync_copy(ntc_h.at[pl.ds(row0, RP)], ntc_v)
        pltpu.sync_copy(bat_h.at[pl.ds(row0, RP)], bat_v)

        def zb(j, carry):
            accs[pl.ds(j * 16, 16)] = _z16f()
            accm[pl.ds(j * 16, 16)] = _z16f()
            return carry
        lax.fori_loop(0, CL // 16, zb, 0)
        for b in range(B):
            bs[pl.ds(b * 16, 16)] = _z16f()
            bm[pl.ds(b * 16, 16)] = _z16f()

        def chunk(ch, carry):
            pltpu.sync_copy(h_h.at[fc, pl.ds(row0 + ch * HCH, HCH)], hbuf)

            def body(g, carry2):
                c16 = ntc_v[pl.ds(ch * HCH + g * 16, 16)]
                b16 = bat_v[pl.ds(ch * HCH + g * 16, 16)]
                for kk in range(16):
                    c = c16[kk] * 16
                    b = b16[kk] * 16
                    row = hbuf[g * 16 + kk, :]
                    accs[pl.ds(c, 16)] = accs[pl.ds(c, 16)] + row
                    accm[pl.ds(c, 16)] = jnp.maximum(accm[pl.ds(c, 16)], row)
                    bs[pl.ds(b, 16)] = bs[pl.ds(b, 16)] + row
                    bm[pl.ds(b, 16)] = jnp.maximum(bm[pl.ds(b, 16)], row)
                return carry2
            lax.fori_loop(0, HCH // 16, body, 0)
            return carry
        lax.fori_loop(0, RP // HCH, chunk, 0)

        @pl.when(nr == 1)
        def _():
            pltpu.sync_copy(accs, st_s.at[fc, 0])
            pltpu.sync_copy(bs, st_bs.at[fc, 0])
            pltpu.sync_copy(bm, st_bm.at[fc, 0])
        plsc.subcore_barrier()

        QL = CL // 4

        @pl.when(nr == 0)
        def _():
            for q in range(4):
                pltpu.sync_copy(st_s.at[fc, 0, pl.ds(q * QL, QL)], cmb)

                def cb(j, carry):
                    accs[pl.ds(q * QL + j * 16, 16)] = (
                        accs[pl.ds(q * QL + j * 16, 16)]
                        + cmb[pl.ds(j * 16, 16)])
                    return carry
                lax.fori_loop(0, QL // 16, cb, 0)
        plsc.subcore_barrier()

        @pl.when(nr == 1)
        def _():
            pltpu.sync_copy(accm, st_s.at[fc, 0])
        plsc.subcore_barrier()

        @pl.when(nr == 0)
        def _():
            for q in range(4):
                pltpu.sync_copy(st_s.at[fc, 0, pl.ds(q * QL, QL)], cmb)

                def cb2(j, carry):
                    accm[pl.ds(q * QL + j * 16, 16)] = jnp.maximum(
                        accm[pl.ds(q * QL + j * 16, 16)],
                        cmb[pl.ds(j * 16, 16)])
                    return carry
                lax.fori_loop(0, QL // 16, cb2, 0)
            pltpu.sync_copy(st_bs.at[fc, 0], cmb.at[pl.ds(0, BL)])
            for b in range(B):
                bs[pl.ds(b * 16, 16)] = (
                    bs[pl.ds(b * 16, 16)] + cmb[pl.ds(b * 16, 16)])
            pltpu.sync_copy(st_bm.at[fc, 0], cmb.at[pl.ds(0, BL)])
            for b in range(B):
                bm[pl.ds(b * 16, 16)] = jnp.maximum(
                    bm[pl.ds(b * 16, 16)], cmb[pl.ds(b * 16, 16)])
            pltpu.sync_copy(accs, xadd_h.at[cid, fc, 0])
            pltpu.sync_copy(accm, xmax_h.at[cid, fc, 0])
            pltpu.sync_copy(bs, bsum_h.at[cid, fc, 0])
            pltpu.sync_copy(bm, bmax_h.at[cid, fc, 0])

    return k(h_fc, ntc, batch)


def _sc_pool2(h2_fc, cbp):
    RP = CPX // 4
    HCH = 128
    BL = B * 16

    @functools.partial(
        pl.kernel,
        out_type=[
            jax.ShapeDtypeStruct((2, 8, 1, BL), jnp.float32),
            jax.ShapeDtypeStruct((2, 8, 1, BL), jnp.float32),
        ],
        mesh=_mesh,
        scratch_types=[
            pltpu.VMEM((HCH, 16), jnp.float32),
            pltpu.VMEM((RP,), jnp.int32),
            pltpu.VMEM((RP,), jnp.int32),
            pltpu.VMEM((BL,), jnp.float32),
            pltpu.VMEM((BL,), jnp.float32),
            pltpu.VMEM((BL,), jnp.float32),
            pltpu.MemorySpace.VMEM_SHARED((8, 1, BL), jnp.float32),
            pltpu.MemorySpace.VMEM_SHARED((8, 1, BL), jnp.float32),
        ],
    )
    def k(h_h, cb_h, bsum_h, bmax_h, hbuf, s0_v, s1_v, bs, bm, cmb, st_bs,
          st_bm):
        cid = lax.axis_index("c")
        sid = lax.axis_index("s")
        nr = sid // 8
        fc = sid % 8
        row0 = cid * (2 * RP) + nr * RP
        pltpu.sync_copy(cb_h.at[0, 0, pl.ds(row0, RP)], s0_v)
        pltpu.sync_copy(cb_h.at[1, 0, pl.ds(row0, RP)], s1_v)

        def mseg(j, carry):
            m = jnp.maximum(s0_v[pl.ds(j * 16, 16)], s1_v[pl.ds(j * 16, 16)])
            s0_v[pl.ds(j * 16, 16)] = jnp.clip(m, 0, B - 1)
            return carry
        lax.fori_loop(0, RP // 16, mseg, 0)
        for b in range(B):
            bs[pl.ds(b * 16, 16)] = _z16f()
            bm[pl.ds(b * 16, 16)] = _z16f()

        def chunk(ch, carry):
            pltpu.sync_copy(h_h.at[fc, pl.ds(row0 + ch * HCH, HCH)], hbuf)

            def body(g, carry2):
                b16 = s0_v[pl.ds(ch * HCH + g * 16, 16)]
                for kk in range(16):
                    b = b16[kk] * 16
                    row = hbuf[g * 16 + kk, :]
                    bs[pl.ds(b, 16)] = bs[pl.ds(b, 16)] + row
                    bm[pl.ds(b, 16)] = jnp.maximum(bm[pl.ds(b, 16)], row)
                return carry2
            lax.fori_loop(0, HCH // 16, body, 0)
            return carry
        lax.fori_loop(0, RP // HCH, chunk, 0)

        @pl.when(nr == 1)
        def _():
            pltpu.sync_copy(bs, st_bs.at[fc, 0])
            pltpu.sync_copy(bm, st_bm.at[fc, 0])
        plsc.subcore_barrier()

        @pl.when(nr == 0)
        def _():
            pltpu.sync_copy(st_bs.at[fc, 0], cmb)
            for b in range(B):
                bs[pl.ds(b * 16, 16)] = (
                    bs[pl.ds(b * 16, 16)] + cmb[pl.ds(b * 16, 16)])
            pltpu.sync_copy(st_bm.at[fc, 0], cmb)
            for b in range(B):
                bm[pl.ds(b * 16, 16)] = jnp.maximum(
                    bm[pl.ds(b * 16, 16)], cmb[pl.ds(b * 16, 16)])
            pltpu.sync_copy(bs, bsum_h.at[cid, fc, 0])
            pltpu.sync_copy(bm, bmax_h.at[cid, fc, 0])

    return k(h2_fc, cbp)


def _dinv_of(degT_blk):
    d = degT_blk[:, 0] + degT_blk[:, 1] + 1.0
    return lax.rsqrt(jnp.clip(d, 1e-12, None))


def _tc_mm_scale(x, W, degT, npad):
    def body(x_ref, w_ref, deg_ref, h_ref, g_ref):
        h = jnp.dot(x_ref[...], w_ref[...], preferred_element_type=jnp.float32)
        dinv = _dinv_of(deg_ref[...])
        h_ref[...] = h
        g_ref[...] = dinv[:, None] * h

    grid = npad // RB
    return pl.pallas_call(
        body,
        grid=(grid,),
        in_specs=[
            pl.BlockSpec((RB, H), lambda i: (i, 0)),
            pl.BlockSpec((H, H), lambda i: (0, 0)),
            pl.BlockSpec((RB, 2), lambda i: (i, 0)),
        ],
        out_specs=[
            pl.BlockSpec((RB, H), lambda i: (i, 0)),
            pl.BlockSpec((RB, H), lambda i: (i, 0)),
        ],
        out_shape=[
            jax.ShapeDtypeStruct((npad, H), jnp.float32),
            jax.ShapeDtypeStruct((npad, H), jnp.float32),
        ],
    )(x, W, degT)


def _tc_gcn_epilogue(p0, p1, hprev, degT, bias, W2, npad, make_g):
    def body(p0_ref, p1_ref, h_ref, deg_ref, b_ref, w_ref,
             x1_ref, h1_ref, g1_ref):
        dinv = _dinv_of(deg_ref[...])
        agg = p0_ref[...] + p1_ref[...]
        x1 = jnp.maximum(
            dinv[:, None] * agg + (dinv * dinv)[:, None] * h_ref[...]
            + b_ref[...], 0.0)
        x1_ref[...] = x1
        h1 = jnp.dot(x1, w_ref[...], preferred_element_type=jnp.float32)
        h1_ref[...] = h1
        g1_ref[...] = dinv[:, None] * h1

    grid = npad // RB
    return pl.pallas_call(
        body,
        grid=(grid,),
        in_specs=[
            pl.BlockSpec((RB, H), lambda i: (i, 0)),
            pl.BlockSpec((RB, H), lambda i: (i, 0)),
            pl.BlockSpec((RB, H), lambda i: (i, 0)),
            pl.BlockSpec((RB, 2), lambda i: (i, 0)),
            pl.BlockSpec((1, H), lambda i: (0, 0)),
            pl.BlockSpec((H, H), lambda i: (0, 0)),
        ],
        out_specs=[
            pl.BlockSpec((RB, H), lambda i: (i, 0)),
            pl.BlockSpec((RB, H), lambda i: (i, 0)),
            pl.BlockSpec((RB, H), lambda i: (i, 0)),
        ],
        out_shape=[
            jax.ShapeDtypeStruct((npad, H), jnp.float32),
            jax.ShapeDtypeStruct((npad, H), jnp.float32),
            jax.ShapeDtypeStruct((npad, H), jnp.float32),
        ],
    )(p0, p1, hprev, degT, bias, W2)


def _tc_block_out(p0, p1, hprev, degT, bias, x1, Wla, Wlb, bl, npad, nvalid):
    def body(p0_ref, p1_ref, h_ref, deg_ref, b_ref, x1_ref, wa_ref, wb_ref,
             bl_ref, out_ref):
        i = pl.program_id(0)
        dinv = _dinv_of(deg_ref[...])
        agg = p0_ref[...] + p1_ref[...]
        x2 = jnp.maximum(
            dinv[:, None] * agg + (dinv * dinv)[:, None] * h_ref[...]
            + b_ref[...], 0.0)
        h = jnp.maximum(
            jnp.dot(x1_ref[...], wa_ref[...], preferred_element_type=jnp.float32)
            + jnp.dot(x2, wb_ref[...], preferred_element_type=jnp.float32)
            + bl_ref[...], 0.0)
        rows = i * RB + lax.broadcasted_iota(jnp.int32, (RB, 1), 0)
        out_ref[...] = jnp.where(rows < nvalid, h, 0.0)

    grid = npad // RB
    return pl.pallas_call(
        body,
        grid=(grid,),
        in_specs=[
            pl.BlockSpec((RB, H), lambda i: (i, 0)),
            pl.BlockSpec((RB, H), lambda i: (i, 0)),
            pl.BlockSpec((RB, H), lambda i: (i, 0)),
            pl.BlockSpec((RB, 2), lambda i: (i, 0)),
            pl.BlockSpec((1, H), lambda i: (0, 0)),
            pl.BlockSpec((RB, H), lambda i: (i, 0)),
            pl.BlockSpec((H, H), lambda i: (0, 0)),
            pl.BlockSpec((H, H), lambda i: (0, 0)),
            pl.BlockSpec((1, H), lambda i: (0, 0)),
        ],
        out_specs=pl.BlockSpec((RB, H), lambda i: (i, 0)),
        out_shape=jax.ShapeDtypeStruct((npad, H), jnp.float32),
    )(p0, p1, hprev, degT, bias, x1, Wla, Wlb, bl)


def _tc_cluster_in(xa0, xa1, xm0, xm1, degT, Ba, Bb):
    def body(a0_ref, a1_ref, m0_ref, m1_ref, deg_ref, ba_ref, bb_ref,
             h_ref, g_ref):
        dinv = _dinv_of(deg_ref[...])
        xadd = a0_ref[...] + a1_ref[...]
        xmax = jnp.maximum(m0_ref[...], m1_ref[...])
        h = (jnp.dot(xadd, ba_ref[...], preferred_element_type=jnp.float32)
             + jnp.dot(xmax, bb_ref[...], preferred_element_type=jnp.float32))
        h_ref[...] = h
        g_ref[...] = dinv[:, None] * h

    grid = CPX // RB
    return pl.pallas_call(
        body,
        grid=(grid,),
        in_specs=[
            pl.BlockSpec((RB, H), lambda i: (i, 0)),
            pl.BlockSpec((RB, H), lambda i: (i, 0)),
            pl.BlockSpec((RB, H), lambda i: (i, 0)),
            pl.BlockSpec((RB, H), lambda i: (i, 0)),
            pl.BlockSpec((RB, 2), lambda i: (i, 0)),
            pl.BlockSpec((H, H), lambda i: (0, 0)),
            pl.BlockSpec((H, H), lambda i: (0, 0)),
        ],
        out_specs=[
            pl.BlockSpec((RB, H), lambda i: (i, 0)),
            pl.BlockSpec((RB, H), lambda i: (i, 0)),
        ],
        out_shape=[
            jax.ShapeDtypeStruct((CPX, H), jnp.float32),
            jax.ShapeDtypeStruct((CPX, H), jnp.float32),
        ],
    )(xa0, xa1, xm0, xm1, degT, Ba, Bb)


def _tc_head(bs1p, bm1p, bs2p, bm2p, bn_g, bn_b, l1W, l1b, l2W, l2b):
    def body(s1_ref, m1_ref, s2_ref, m2_ref, g_ref, b_ref, w1_ref, b1_ref,
             w2_ref, b2_ref, out_ref):
        bs1 = s1_ref[0] + s1_ref[1]
        bm1 = jnp.maximum(m1_ref[0], m1_ref[1])
        bs2 = s2_ref[0] + s2_ref[1]
        bm2 = jnp.maximum(m2_ref[0], m2_ref[1])
        z = jnp.concatenate([bs1, bm1, bs2, bm2], axis=1)
        mean = jnp.mean(z, axis=0, keepdims=True)
        var = jnp.mean((z - mean) ** 2, axis=0, keepdims=True)
        z = (z - mean) / jnp.sqrt(var + 1e-5) * g_ref[...] + b_ref[...]
        hh = jnp.maximum(
            jnp.dot(z, w1_ref[...], preferred_element_type=jnp.float32)
            + b1_ref[...], 0.0)
        o = (jnp.dot(hh, w2_ref[...], preferred_element_type=jnp.float32)
             + b2_ref[...])
        o = o - jnp.max(o, axis=-1, keepdims=True)
        e = jnp.exp(o)
        out_ref[...] = e / jnp.sum(e, axis=-1, keepdims=True)

    return pl.pallas_call(
        body,
        out_shape=jax.ShapeDtypeStruct((B, NCLS), jnp.float32),
    )(bs1p, bm1p, bs2p, bm2p, bn_g[None, :], bn_b[None, :], l1W, l1b[None, :],
      l2W, l2b[None, :])


def _fc_layout(h, nvalid, npad):
    return jnp.transpose(h.reshape(npad, 8, 16), (1, 0, 2))


def kernel(x, edge_weight, cW1, cb1, cW2, cb2, cWl, cbl, bW1, bb1, bW2, bb2,
           bWl, bbl, bn_g, bn_b, l1W, l1b, l2W, l2b, edge_index,
           node_to_cluster, batch):
    src, dst = edge_index[0], edge_index[1]
    srcp = jnp.pad(src, (0, EP - E))
    dstp = jnp.pad(dst, (0, EP - E))
    wp = jnp.pad(edge_weight, (0, EP - E))
    ntcp = jnp.pad(node_to_cluster, (0, NP - N))
    batp = jnp.pad(batch, (0, NP - N))
    xp = jnp.pad(x, ((0, NP - N), (0, 0)))

    psrc, pdst, deg1p, deg2p, cbp = _sc_pre(ntcp, batp, srcp, dstp, wp)
    src3 = srcp.reshape(EP // EK, 1, EK)
    dst3 = dstp.reshape(EP // EK, 1, EK)
    w3 = wp.reshape(EP // EK, 1, EK)
    psrc3 = psrc.reshape(EP // EK, 1, EK)
    pdst3 = pdst.reshape(EP // EK, 1, EK)
    deg1T = jnp.transpose(deg1p.reshape(2, NP))
    deg2T = jnp.transpose(deg2p.reshape(2, CPX))

    h0, g0 = _tc_mm_scale(xp, cW1, deg1T, NP)
    a1 = _sc_agg(g0, src3, dst3, w3, NP)
    x1, h1, g1 = _tc_gcn_epilogue(a1[0], a1[1], h0, deg1T, cb1[None, :], cW2,
                                  NP, True)
    a2 = _sc_agg(g1, src3, dst3, w3, NP)
    h = _tc_block_out(a2[0], a2[1], h1, deg1T, cb2[None, :], x1,
                      cWl[:H], cWl[H:], cbl[None, :], NP, N)

    h_fc = _fc_layout(h, N, NP)
    xaddp, xmaxp, bs1p, bm1p = _sc_pool1(h_fc, ntcp, batp)

    def _cl(t):
        return jnp.transpose(
            t.reshape(2, 8, CPX, 16), (0, 2, 1, 3)).reshape(2, CPX, H)

    xa = _cl(xaddp)
    xm = _cl(xmaxp)

    hb0, g2 = _tc_cluster_in(xa[0], xa[1], xm[0], xm[1], deg2T,
                             bW1[:H], bW1[H:])
    a3 = _sc_agg(g2, psrc3, pdst3, w3, CPX)
    y1, h3, g3 = _tc_gcn_epilogue(a3[0], a3[1], hb0, deg2T, bb1[None, :], bW2,
                                  CPX, True)
    a4 = _sc_agg(g3, psrc3, pdst3, w3, CPX)
    h2 = _tc_block_out(a4[0], a4[1], h3, deg2T, bb2[None, :], y1,
                       bWl[:H], bWl[H:], bbl[None, :], CPX, C)

    h2_fc = _fc_layout(h2, C, CPX)
    bs2p, bm2p = _sc_pool2(h2_fc, cbp)

    def _bt(t):
        return jnp.transpose(t.reshape(2, 8, B, 16), (0, 2, 1, 3)).reshape(2, B, H)

    return _tc_head(_bt(bs1p), _bt(bm1p), _bt(bs2p), _bt(bm2p),
                    bn_g, bn_b, l1W, l1b, l2W, l2b)


# speedup vs baseline: 1.4421x; 1.4421x over previous
import functools

import jax
import jax.numpy as jnp
from jax import lax
from jax.experimental import pallas as pl
from jax.experimental.pallas import tpu as pltpu, tpu_sc as plsc

N = 10000
E = 320000
C = 2000
B = 8
H = 128
NCLS = 10

NP = 10240
CPX = 2048
ET = 10112
EP = 32 * ET
EK = 128
NCH = ET // EK
RB = 512

_mesh = plsc.VectorSubcoreMesh(core_axis_name="c", subcore_axis_name="s")


def _z16f():
    return jnp.full((16,), 0.0, dtype=jnp.float32)


def _z16i():
    return jnp.full((16,), 0, dtype=jnp.int32)


def _sc_pre(ntc, batch, srcp, dstp, wp):
    NT = NP // 16

    @functools.partial(
        pl.kernel,
        out_type=[
            jax.ShapeDtypeStruct((EP,), jnp.int32),
            jax.ShapeDtypeStruct((EP,), jnp.int32),
            jax.ShapeDtypeStruct((2, 1, NP), jnp.float32),
            jax.ShapeDtypeStruct((2, 1, CPX), jnp.float32),
            jax.ShapeDtypeStruct((2, 1, CPX), jnp.int32),
        ],
        mesh=_mesh,
        scratch_types=[
            pltpu.VMEM((EK,), jnp.int32),
            pltpu.VMEM((EK,), jnp.int32),
            pltpu.VMEM((EK,), jnp.float32),
            pltpu.VMEM((EK,), jnp.int32),
            pltpu.VMEM((EK,), jnp.int32),
            pltpu.VMEM((NT,), jnp.int32),
            pltpu.VMEM((NT,), jnp.int32),
            pltpu.VMEM((CPX,), jnp.int32),
            pltpu.VMEM((640,), jnp.float32),
            pltpu.MemorySpace.VMEM_SHARED((NP,), jnp.float32),
            pltpu.MemorySpace.VMEM_SHARED((CPX,), jnp.float32),
            pltpu.MemorySpace.VMEM_SHARED((16, 1, CPX), jnp.int32),
            pltpu.SemaphoreType.DMA,
            pltpu.SemaphoreType.DMA,
        ],
    )
    def k(ntc_h, bat_h, src_h, dst_h, w_h, psrc_h, pdst_h, deg1_h, deg2_h, cb_h,
          src_v, dst_v, w_v, ps_v, pd_v, ntc_v, bat_v, cb_v, zv,
          deg1_sh, deg2_sh, cb_st, sem, sem2):
        cid = lax.axis_index("c")
        sid = lax.axis_index("s")
        wid = sid * 2 + cid

        def zb(j, carry):
            zv[pl.ds(j * 16, 16)] = _z16f()
            return carry
        lax.fori_loop(0, 40, zb, 0)
        pltpu.sync_copy(zv, deg1_sh.at[pl.ds(sid * 640, 640)])
        pltpu.sync_copy(zv.at[pl.ds(0, 128)], deg2_sh.at[pl.ds(sid * 128, 128)])

        def zcb(j, carry):
            cb_v[pl.ds(j * 16, 16)] = _z16i()
            return carry
        lax.fori_loop(0, CPX // 16, zcb, 0)

        nbase = wid * (NP // 32)
        pltpu.sync_copy(ntc_h.at[pl.ds(nbase, 320)], ntc_v.at[pl.ds(0, 320)])
        pltpu.sync_copy(bat_h.at[pl.ds(nbase, 320)], bat_v.at[pl.ds(0, 320)])
        lanes = lax.iota(jnp.int32, 16)

        def nb(g, carry):
            c16 = ntc_v[pl.ds(g * 16, 16)]
            b16 = bat_v[pl.ds(g * 16, 16)]
            for kk in range(16):
                c = c16[kk]
                b = b16[kk]
                ca = (c >> 4) << 4
                lane = c - ca
                cur = cb_v[pl.ds(ca, 16)]
                cb_v[pl.ds(ca, 16)] = jnp.where(
                    lanes == lane, jnp.maximum(cur, b), cur)
            return carry
        lax.fori_loop(0, 20, nb, 0)
        pltpu.sync_copy(cb_v, cb_st.at[sid, 0])
        plsc.subcore_barrier()

        ebase = wid * ET

        def eb(ch, carry):
            off = ebase + ch * EK
            pltpu.sync_copy(src_h.at[pl.ds(off, EK)], src_v)
            pltpu.sync_copy(dst_h.at[pl.ds(off, EK)], dst_v)
            pltpu.sync_copy(w_h.at[pl.ds(off, EK)], w_v)
            cp1 = pltpu.async_copy(ntc_h.at[src_v], ps_v, sem)
            cp2 = pltpu.async_copy(ntc_h.at[dst_v], pd_v, sem2)
            cp1.wait()
            cp2.wait()
            pltpu.sync_copy(ps_v, psrc_h.at[pl.ds(off, EK)])
            pltpu.sync_copy(pd_v, pdst_h.at[pl.ds(off, EK)])
            pltpu.sync_copy(w_v, deg1_sh.at[dst_v], add=True)
            pltpu.sync_copy(w_v, deg2_sh.at[pd_v], add=True)
            return carry
        lax.fori_loop(0, NCH, eb, 0)
        plsc.subcore_barrier()

        cslice = sid * 128
        pltpu.sync_copy(cb_st.at[0, 0, pl.ds(cslice, 128)],
                        cb_v.at[pl.ds(0, 128)])
        for t in range(1, 16):
            pltpu.sync_copy(
                cb_st.at[t, 0, pl.ds(cslice, 128)], cb_v.at[pl.ds(128, 128)])
            for j in range(0, 128, 16):
                cb_v[pl.ds(j, 16)] = jnp.maximum(
                    cb_v[pl.ds(j, 16)], cb_v[pl.ds(128 + j, 16)])
        pltpu.sync_copy(cb_v.at[pl.ds(0, 128)],
                        cb_h.at[cid, 0, pl.ds(cslice, 128)])
        pltpu.sync_copy(deg1_sh.at[pl.ds(sid * 640, 640)],
                        deg1_h.at[cid, 0, pl.ds(sid * 640, 640)])
        pltpu.sync_copy(deg2_sh.at[pl.ds(sid * 128, 128)],
                        deg2_h.at[cid, 0, pl.ds(sid * 128, 128)])

    return k(ntc, batch, srcp, dstp, wp)


def _sc_agg(table, edata, wp, npad):
    RT = npad // 16

    @functools.partial(
        pl.kernel,
        out_type=jax.ShapeDtypeStruct((2, npad, H), jnp.float32),
        mesh=_mesh,
        scratch_types=[
            pltpu.VMEM((2, 2, EK), jnp.int32),
            pltpu.VMEM((2, EK), jnp.float32),
            pltpu.VMEM((2, EK, H), jnp.float32),
            pltpu.VMEM((16, H), jnp.float32),
            pltpu.MemorySpace.VMEM_SHARED((npad, H), jnp.float32),
            pltpu.SemaphoreType.DMA,
            pltpu.SemaphoreType.DMA,
        ],
    )
    def k(tab_h, ed_h, w_h, out_h, ed_v, w_v, rows_v, zbuf, acc_sh, sg0, sg1):
        cid = lax.axis_index("c")
        sid = lax.axis_index("s")
        wid = sid * 2 + cid
        for r in range(16):
            for j in range(0, H, 16):
                zbuf[r, pl.ds(j, 16)] = _z16f()

        def zb(z, carry):
            pltpu.sync_copy(zbuf, acc_sh.at[pl.ds(sid * RT + z * 16, 16)])
            return carry
        lax.fori_loop(0, RT // 16, zb, 0)
        plsc.subcore_barrier()

        cbase = wid * NCH
        sems = (sg0, sg1)

        def load_and_gather(ch, slot):
            pltpu.sync_copy(ed_h.at[cbase + ch], ed_v.at[slot])
            pltpu.sync_copy(
                w_h.at[pl.ds((cbase + ch) * EK, EK)], w_v.at[slot])
            pltpu.async_copy(tab_h.at[ed_v.at[slot, 0]], rows_v.at[slot],
                             sems[slot])

        def consume(slot):
            pltpu.make_async_copy(
                tab_h.at[ed_v.at[slot, 0]], rows_v.at[slot],
                sems[slot]).wait()

            def sc(g, carry2):
                w16 = w_v[slot, pl.ds(g * 16, 16)]
                for kk in range(16):
                    e = g * 16 + kk
                    wk = w16[kk]
                    for j in range(0, H, 16):
                        rows_v[slot, e, pl.ds(j, 16)] = (
                            rows_v[slot, e, pl.ds(j, 16)] * wk)
                return carry2
            lax.fori_loop(0, EK // 16, sc, 0, unroll=2)
            pltpu.sync_copy(rows_v.at[slot], acc_sh.at[ed_v.at[slot, 1]],
                            add=True)

        load_and_gather(0, 0)

        def body(p, carry):
            ch0 = 2 * p
            ch1 = ch0 + 1

            @pl.when(ch1 < NCH)
            def _():
                load_and_gather(ch1, 1)
            consume(0)

            @pl.when(ch0 + 2 < NCH)
            def _():
                load_and_gather(ch0 + 2, 0)

            @pl.when(ch1 < NCH)
            def _():
                consume(1)
            return carry
        lax.fori_loop(0, (NCH + 1) // 2, body, 0)
        plsc.subcore_barrier()
        pltpu.sync_copy(acc_sh.at[pl.ds(sid * RT, RT)],
                        out_h.at[cid, pl.ds(sid * RT, RT)])

    return k(table, edata, wp)


def _sc_pool1(h_fc, ntc, batch):
    RP = NP // 4
    HCH = 128
    CL = CPX * 16
    BL = B * 16

    @functools.partial(
        pl.kernel,
        out_type=[
            jax.ShapeDtypeStruct((2, 8, 1, CL), jnp.float32),
            jax.ShapeDtypeStruct((2, 8, 1, CL), jnp.float32),
            jax.ShapeDtypeStruct((2, 8, 1, BL), jnp.float32),
            jax.ShapeDtypeStruct((2, 8, 1, BL), jnp.float32),
        ],
        mesh=_mesh,
        scratch_types=[
            pltpu.VMEM((HCH, 16), jnp.float32),
            pltpu.VMEM((RP,), jnp.int32),
            pltpu.VMEM((RP,), jnp.int32),
            pltpu.VMEM((CL,), jnp.float32),
            pltpu.VMEM((CL,), jnp.float32),
            pltpu.VMEM((CL // 4,), jnp.float32),
            pltpu.VMEM((BL,), jnp.float32),
            pltpu.VMEM((BL,), jnp.float32),
            pltpu.MemorySpace.VMEM_SHARED((8, 1, CL), jnp.float32),
            pltpu.MemorySpace.VMEM_SHARED((8, 1, BL), jnp.float32),
            pltpu.MemorySpace.VMEM_SHARED((8, 1, BL), jnp.float32),
        ],
    )
    def k(h_h, ntc_h, bat_h, xadd_h, xmax_h, bsum_h, bmax_h,
          hbuf, ntc_v, bat_v, accs, accm, cmb, bs, bm, st_s, st_bs,
          st_bm):
        cid = lax.axis_index("c")
        sid = lax.axis_index("s")
        nr = sid // 8
        fc = sid % 8
        row0 = cid * (2 * RP) + nr * RP
        pltpu.sync_copy(ntc_h.at[pl.ds(row0, RP)], ntc_v)
        pltpu.sync_copy(bat_h.at[pl.ds(row0, RP)], bat_v)

        def zb(j, carry):
            accs[pl.ds(j * 16, 16)] = _z16f()
            accm[pl.ds(j * 16, 16)] = _z16f()
            return carry
        lax.fori_loop(0, CL // 16, zb, 0)
        for b in range(B):
            bs[pl.ds(b * 16, 16)] = _z16f()
            bm[pl.ds(b * 16, 16)] = _z16f()

        def chunk(ch, carry):
            pltpu.sync_copy(h_h.at[fc, pl.ds(row0 + ch * HCH, HCH)], hbuf)

            def body(g, carry2):
                c16 = ntc_v[pl.ds(ch * HCH + g * 16, 16)]
                b16 = bat_v[pl.ds(ch * HCH + g * 16, 16)]
                for kk in range(16):
                    c = c16[kk] * 16
                    b = b16[kk] * 16
                    row = hbuf[g * 16 + kk, :]
                    accs[pl.ds(c, 16)] = accs[pl.ds(c, 16)] + row
                    accm[pl.ds(c, 16)] = jnp.maximum(accm[pl.ds(c, 16)], row)
                    bs[pl.ds(b, 16)] = bs[pl.ds(b, 16)] + row
                    bm[pl.ds(b, 16)] = jnp.maximum(bm[pl.ds(b, 16)], row)
                return carry2
            lax.fori_loop(0, HCH // 16, body, 0)
            return carry
        lax.fori_loop(0, RP // HCH, chunk, 0)

        @pl.when(nr == 1)
        def _():
            pltpu.sync_copy(accs, st_s.at[fc, 0])
            pltpu.sync_copy(bs, st_bs.at[fc, 0])
            pltpu.sync_copy(bm, st_bm.at[fc, 0])
        plsc.subcore_barrier()

        QL = CL // 4

        @pl.when(nr == 0)
        def _():
            for q in range(4):
                pltpu.sync_copy(st_s.at[fc, 0, pl.ds(q * QL, QL)], cmb)

                def cb(j, carry):
                    accs[pl.ds(q * QL + j * 16, 16)] = (
                        accs[pl.ds(q * QL + j * 16, 16)]
                        + cmb[pl.ds(j * 16, 16)])
                    return carry
                lax.fori_loop(0, QL // 16, cb, 0)
        plsc.subcore_barrier()

        @pl.when(nr == 1)
        def _():
            pltpu.sync_copy(accm, st_s.at[fc, 0])
        plsc.subcore_barrier()

        @pl.when(nr == 0)
        def _():
            for q in range(4):
                pltpu.sync_copy(st_s.at[fc, 0, pl.ds(q * QL, QL)], cmb)

                def cb2(j, carry):
                    accm[pl.ds(q * QL + j * 16, 16)] = jnp.maximum(
                        accm[pl.ds(q * QL + j * 16, 16)],
                        cmb[pl.ds(j * 16, 16)])
                    return carry
                lax.fori_loop(0, QL // 16, cb2, 0)
            pltpu.sync_copy(st_bs.at[fc, 0], cmb.at[pl.ds(0, BL)])
            for b in range(B):
                bs[pl.ds(b * 16, 16)] = (
                    bs[pl.ds(b * 16, 16)] + cmb[pl.ds(b * 16, 16)])
            pltpu.sync_copy(st_bm.at[fc, 0], cmb.at[pl.ds(0, BL)])
            for b in range(B):
                bm[pl.ds(b * 16, 16)] = jnp.maximum(
                    bm[pl.ds(b * 16, 16)], cmb[pl.ds(b * 16, 16)])
            pltpu.sync_copy(accs, xadd_h.at[cid, fc, 0])
            pltpu.sync_copy(accm, xmax_h.at[cid, fc, 0])
            pltpu.sync_copy(bs, bsum_h.at[cid, fc, 0])
            pltpu.sync_copy(bm, bmax_h.at[cid, fc, 0])

    return k(h_fc, ntc, batch)


def _sc_pool2(h2_fc, cbp):
    RP = CPX // 4
    HCH = 128
    BL = B * 16

    @functools.partial(
        pl.kernel,
        out_type=[
            jax.ShapeDtypeStruct((2, 8, 1, BL), jnp.float32),
            jax.ShapeDtypeStruct((2, 8, 1, BL), jnp.float32),
        ],
        mesh=_mesh,
        scratch_types=[
            pltpu.VMEM((HCH, 16), jnp.float32),
            pltpu.VMEM((RP,), jnp.int32),
            pltpu.VMEM((RP,), jnp.int32),
            pltpu.VMEM((BL,), jnp.float32),
            pltpu.VMEM((BL,), jnp.float32),
            pltpu.VMEM((BL,), jnp.float32),
            pltpu.MemorySpace.VMEM_SHARED((8, 1, BL), jnp.float32),
            pltpu.MemorySpace.VMEM_SHARED((8, 1, BL), jnp.float32),
        ],
    )
    def k(h_h, cb_h, bsum_h, bmax_h, hbuf, s0_v, s1_v, bs, bm, cmb, st_bs,
          st_bm):
        cid = lax.axis_index("c")
        sid = lax.axis_index("s")
        nr = sid // 8
        fc = sid % 8
        row0 = cid * (2 * RP) + nr * RP
        pltpu.sync_copy(cb_h.at[0, 0, pl.ds(row0, RP)], s0_v)
        pltpu.sync_copy(cb_h.at[1, 0, pl.ds(row0, RP)], s1_v)

        def mseg(j, carry):
            m = jnp.maximum(s0_v[pl.ds(j * 16, 16)], s1_v[pl.ds(j * 16, 16)])
            s0_v[pl.ds(j * 16, 16)] = jnp.clip(m, 0, B - 1)
            return carry
        lax.fori_loop(0, RP // 16, mseg, 0)
        for b in range(B):
            bs[pl.ds(b * 16, 16)] = _z16f()
            bm[pl.ds(b * 16, 16)] = _z16f()

        def chunk(ch, carry):
            pltpu.sync_copy(h_h.at[fc, pl.ds(row0 + ch * HCH, HCH)], hbuf)

            def body(g, carry2):
                b16 = s0_v[pl.ds(ch * HCH + g * 16, 16)]
                for kk in range(16):
                    b = b16[kk] * 16
                    row = hbuf[g * 16 + kk, :]
                    bs[pl.ds(b, 16)] = bs[pl.ds(b, 16)] + row
                    bm[pl.ds(b, 16)] = jnp.maximum(bm[pl.ds(b, 16)], row)
                return carry2
            lax.fori_loop(0, HCH // 16, body, 0)
            return carry
        lax.fori_loop(0, RP // HCH, chunk, 0)

        @pl.when(nr == 1)
        def _():
            pltpu.sync_copy(bs, st_bs.at[fc, 0])
            pltpu.sync_copy(bm, st_bm.at[fc, 0])
        plsc.subcore_barrier()

        @pl.when(nr == 0)
        def _():
            pltpu.sync_copy(st_bs.at[fc, 0], cmb)
            for b in range(B):
                bs[pl.ds(b * 16, 16)] = (
                    bs[pl.ds(b * 16, 16)] + cmb[pl.ds(b * 16, 16)])
            pltpu.sync_copy(st_bm.at[fc, 0], cmb)
            for b in range(B):
                bm[pl.ds(b * 16, 16)] = jnp.maximum(
                    bm[pl.ds(b * 16, 16)], cmb[pl.ds(b * 16, 16)])
            pltpu.sync_copy(bs, bsum_h.at[cid, fc, 0])
            pltpu.sync_copy(bm, bmax_h.at[cid, fc, 0])

    return k(h2_fc, cbp)


def _dinv_of(degT_blk):
    d = degT_blk[:, 0] + degT_blk[:, 1] + 1.0
    return lax.rsqrt(jnp.clip(d, 1e-12, None))


def _tc_mm_scale(x, W, degT, npad):
    def body(x_ref, w_ref, deg_ref, h_ref, g_ref):
        h = jnp.dot(x_ref[...], w_ref[...], preferred_element_type=jnp.float32)
        dinv = _dinv_of(deg_ref[...])
        h_ref[...] = h
        g_ref[...] = dinv[:, None] * h

    grid = npad // RB
    return pl.pallas_call(
        body,
        grid=(grid,),
        in_specs=[
            pl.BlockSpec((RB, H), lambda i: (i, 0)),
            pl.BlockSpec((H, H), lambda i: (0, 0)),
            pl.BlockSpec((RB, 2), lambda i: (i, 0)),
        ],
        out_specs=[
            pl.BlockSpec((RB, H), lambda i: (i, 0)),
            pl.BlockSpec((RB, H), lambda i: (i, 0)),
        ],
        out_shape=[
            jax.ShapeDtypeStruct((npad, H), jnp.float32),
            jax.ShapeDtypeStruct((npad, H), jnp.float32),
        ],
    )(x, W, degT)


def _tc_gcn_epilogue(p0, p1, hprev, degT, bias, W2, npad, make_g):
    def body(p0_ref, p1_ref, h_ref, deg_ref, b_ref, w_ref,
             x1_ref, h1_ref, g1_ref):
        dinv = _dinv_of(deg_ref[...])
        agg = p0_ref[...] + p1_ref[...]
        x1 = jnp.maximum(
            dinv[:, None] * agg + (dinv * dinv)[:, None] * h_ref[...]
            + b_ref[...], 0.0)
        x1_ref[...] = x1
        h1 = jnp.dot(x1, w_ref[...], preferred_element_type=jnp.float32)
        h1_ref[...] = h1
        g1_ref[...] = dinv[:, None] * h1

    grid = npad // RB
    return pl.pallas_call(
        body,
        grid=(grid,),
        in_specs=[
            pl.BlockSpec((RB, H), lambda i: (i, 0)),
            pl.BlockSpec((RB, H), lambda i: (i, 0)),
            pl.BlockSpec((RB, H), lambda i: (i, 0)),
            pl.BlockSpec((RB, 2), lambda i: (i, 0)),
            pl.BlockSpec((1, H), lambda i: (0, 0)),
            pl.BlockSpec((H, H), lambda i: (0, 0)),
        ],
        out_specs=[
            pl.BlockSpec((RB, H), lambda i: (i, 0)),
            pl.BlockSpec((RB, H), lambda i: (i, 0)),
            pl.BlockSpec((RB, H), lambda i: (i, 0)),
        ],
        out_shape=[
            jax.ShapeDtypeStruct((npad, H), jnp.float32),
            jax.ShapeDtypeStruct((npad, H), jnp.float32),
            jax.ShapeDtypeStruct((npad, H), jnp.float32),
        ],
    )(p0, p1, hprev, degT, bias, W2)


def _tc_block_out(p0, p1, hprev, degT, bias, x1, Wla, Wlb, bl, npad, nvalid):
    def body(p0_ref, p1_ref, h_ref, deg_ref, b_ref, x1_ref, wa_ref, wb_ref,
             bl_ref, out_ref):
        i = pl.program_id(0)
        dinv = _dinv_of(deg_ref[...])
        agg = p0_ref[...] + p1_ref[...]
        x2 = jnp.maximum(
            dinv[:, None] * agg + (dinv * dinv)[:, None] * h_ref[...]
            + b_ref[...], 0.0)
        h = jnp.maximum(
            jnp.dot(x1_ref[...], wa_ref[...], preferred_element_type=jnp.float32)
            + jnp.dot(x2, wb_ref[...], preferred_element_type=jnp.float32)
            + bl_ref[...], 0.0)
        rows = i * RB + lax.broadcasted_iota(jnp.int32, (RB, 1), 0)
        out_ref[...] = jnp.where(rows < nvalid, h, 0.0)

    grid = npad // RB
    return pl.pallas_call(
        body,
        grid=(grid,),
        in_specs=[
            pl.BlockSpec((RB, H), lambda i: (i, 0)),
            pl.BlockSpec((RB, H), lambda i: (i, 0)),
            pl.BlockSpec((RB, H), lambda i: (i, 0)),
            pl.BlockSpec((RB, 2), lambda i: (i, 0)),
            pl.BlockSpec((1, H), lambda i: (0, 0)),
            pl.BlockSpec((RB, H), lambda i: (i, 0)),
            pl.BlockSpec((H, H), lambda i: (0, 0)),
            pl.BlockSpec((H, H), lambda i: (0, 0)),
            pl.BlockSpec((1, H), lambda i: (0, 0)),
        ],
        out_specs=pl.BlockSpec((RB, H), lambda i: (i, 0)),
        out_shape=jax.ShapeDtypeStruct((npad, H), jnp.float32),
    )(p0, p1, hprev, degT, bias, x1, Wla, Wlb, bl)


def _tc_cluster_in(xa0, xa1, xm0, xm1, degT, Ba, Bb):
    def body(a0_ref, a1_ref, m0_ref, m1_ref, deg_ref, ba_ref, bb_ref,
             h_ref, g_ref):
        dinv = _dinv_of(deg_ref[...])
        xadd = a0_ref[...] + a1_ref[...]
        xmax = jnp.maximum(m0_ref[...], m1_ref[...])
        h = (jnp.dot(xadd, ba_ref[...], preferred_element_type=jnp.float32)
             + jnp.dot(xmax, bb_ref[...], preferred_element_type=jnp.float32))
        h_ref[...] = h
        g_ref[...] = dinv[:, None] * h

    grid = CPX // RB
    return pl.pallas_call(
        body,
        grid=(grid,),
        in_specs=[
            pl.BlockSpec((RB, H), lambda i: (i, 0)),
            pl.BlockSpec((RB, H), lambda i: (i, 0)),
            pl.BlockSpec((RB, H), lambda i: (i, 0)),
            pl.BlockSpec((RB, H), lambda i: (i, 0)),
            pl.BlockSpec((RB, 2), lambda i: (i, 0)),
            pl.BlockSpec((H, H), lambda i: (0, 0)),
            pl.BlockSpec((H, H), lambda i: (0, 0)),
        ],
        out_specs=[
            pl.BlockSpec((RB, H), lambda i: (i, 0)),
            pl.BlockSpec((RB, H), lambda i: (i, 0)),
        ],
        out_shape=[
            jax.ShapeDtypeStruct((CPX, H), jnp.float32),
            jax.ShapeDtypeStruct((CPX, H), jnp.float32),
        ],
    )(xa0, xa1, xm0, xm1, degT, Ba, Bb)


def _tc_head(bs1p, bm1p, bs2p, bm2p, bn_g, bn_b, l1W, l1b, l2W, l2b):
    def body(s1_ref, m1_ref, s2_ref, m2_ref, g_ref, b_ref, w1_ref, b1_ref,
             w2_ref, b2_ref, out_ref):
        bs1 = s1_ref[0] + s1_ref[1]
        bm1 = jnp.maximum(m1_ref[0], m1_ref[1])
        bs2 = s2_ref[0] + s2_ref[1]
        bm2 = jnp.maximum(m2_ref[0], m2_ref[1])
        z = jnp.concatenate([bs1, bm1, bs2, bm2], axis=1)
        mean = jnp.mean(z, axis=0, keepdims=True)
        var = jnp.mean((z - mean) ** 2, axis=0, keepdims=True)
        z = (z - mean) / jnp.sqrt(var + 1e-5) * g_ref[...] + b_ref[...]
        hh = jnp.maximum(
            jnp.dot(z, w1_ref[...], preferred_element_type=jnp.float32)
            + b1_ref[...], 0.0)
        o = (jnp.dot(hh, w2_ref[...], preferred_element_type=jnp.float32)
             + b2_ref[...])
        o = o - jnp.max(o, axis=-1, keepdims=True)
        e = jnp.exp(o)
        out_ref[...] = e / jnp.sum(e, axis=-1, keepdims=True)

    return pl.pallas_call(
        body,
        out_shape=jax.ShapeDtypeStruct((B, NCLS), jnp.float32),
    )(bs1p, bm1p, bs2p, bm2p, bn_g[None, :], bn_b[None, :], l1W, l1b[None, :],
      l2W, l2b[None, :])


def _fc_layout(h, nvalid, npad):
    return jnp.transpose(h.reshape(npad, 8, 16), (1, 0, 2))


def kernel(x, edge_weight, cW1, cb1, cW2, cb2, cWl, cbl, bW1, bb1, bW2, bb2,
           bWl, bbl, bn_g, bn_b, l1W, l1b, l2W, l2b, edge_index,
           node_to_cluster, batch):
    src, dst = edge_index[0], edge_index[1]
    srcp = jnp.pad(src, (0, EP - E))
    dstp = jnp.pad(dst, (0, EP - E))
    wp = jnp.pad(edge_weight, (0, EP - E))
    ntcp = jnp.pad(node_to_cluster, (0, NP - N))
    batp = jnp.pad(batch, (0, NP - N))
    xp = jnp.pad(x, ((0, NP - N), (0, 0)))

    psrc, pdst, deg1p, deg2p, cbp = _sc_pre(ntcp, batp, srcp, dstp, wp)
    ed1 = jnp.concatenate(
        [srcp.reshape(EP // EK, 1, EK), dstp.reshape(EP // EK, 1, EK)], axis=1)
    ed2 = jnp.concatenate(
        [psrc.reshape(EP // EK, 1, EK), pdst.reshape(EP // EK, 1, EK)], axis=1)
    deg1T = jnp.transpose(deg1p.reshape(2, NP))
    deg2T = jnp.transpose(deg2p.reshape(2, CPX))

    h0, g0 = _tc_mm_scale(xp, cW1, deg1T, NP)
    a1 = _sc_agg(g0, ed1, wp, NP)
    x1, h1, g1 = _tc_gcn_epilogue(a1[0], a1[1], h0, deg1T, cb1[None, :], cW2,
                                  NP, True)
    a2 = _sc_agg(g1, ed1, wp, NP)
    h = _tc_block_out(a2[0], a2[1], h1, deg1T, cb2[None, :], x1,
                      cWl[:H], cWl[H:], cbl[None, :], NP, N)

    h_fc = _fc_layout(h, N, NP)
    xaddp, xmaxp, bs1p, bm1p = _sc_pool1(h_fc, ntcp, batp)

    def _cl(t):
        return jnp.transpose(
            t.reshape(2, 8, CPX, 16), (0, 2, 1, 3)).reshape(2, CPX, H)

    xa = _cl(xaddp)
    xm = _cl(xmaxp)

    hb0, g2 = _tc_cluster_in(xa[0], xa[1], xm[0], xm[1], deg2T,
                             bW1[:H], bW1[H:])
    a3 = _sc_agg(g2, ed2, wp, CPX)
    y1, h3, g3 = _tc_gcn_epilogue(a3[0], a3[1], hb0, deg2T, bb1[None, :], bW2,
                                  CPX, True)
    a4 = _sc_agg(g3, ed2, wp, CPX)
    h2 = _tc_block_out(a4[0], a4[1], h3, deg2T, bb2[None, :], y1,
                       bWl[:H], bWl[H:], bbl[None, :], CPX, C)

    h2_fc = _fc_layout(h2, C, CPX)
    bs2p, bm2p = _sc_pool2(h2_fc, cbp)

    def _bt(t):
        return jnp.transpose(t.reshape(2, 8, B, 16), (0, 2, 1, 3)).reshape(2, B, H)

    return _tc_head(_bt(bs1p), _bt(bm1p), _bt(bs2p), _bt(bm2p),
                    bn_g, bn_b, l1W, l1b, l2W, l2b)


# speedup vs baseline: 1.4460x; 1.0027x over previous
import functools

import jax
import jax.numpy as jnp
from jax import lax
from jax.experimental import pallas as pl
from jax.experimental.pallas import tpu as pltpu, tpu_sc as plsc

N = 10000
E = 320000
C = 2000
B = 8
H = 128
NCLS = 10

NP = 10240
CPX = 2048
ET = 10112
EP = 32 * ET
EK = 128
NCH = ET // EK
RB = 512

_mesh = plsc.VectorSubcoreMesh(core_axis_name="c", subcore_axis_name="s")


def _z16f():
    return jnp.full((16,), 0.0, dtype=jnp.float32)


def _z16i():
    return jnp.full((16,), 0, dtype=jnp.int32)


def _sc_pre(ntc, batch, ed, wp):
    NT = NP // 16

    @functools.partial(
        pl.kernel,
        out_type=[
            jax.ShapeDtypeStruct((EP,), jnp.int32),
            jax.ShapeDtypeStruct((EP,), jnp.int32),
            jax.ShapeDtypeStruct((2, 1, NP), jnp.float32),
            jax.ShapeDtypeStruct((2, 1, CPX), jnp.float32),
            jax.ShapeDtypeStruct((2, 1, CPX), jnp.int32),
        ],
        mesh=_mesh,
        scratch_types=[
            pltpu.VMEM((2, EK), jnp.int32),
            pltpu.VMEM((EK,), jnp.float32),
            pltpu.VMEM((2, EK), jnp.int32),
            pltpu.VMEM((2, EK), jnp.int32),
            pltpu.VMEM((NT,), jnp.int32),
            pltpu.VMEM((NT,), jnp.int32),
            pltpu.VMEM((CPX,), jnp.int32),
            pltpu.VMEM((640,), jnp.float32),
            pltpu.MemorySpace.VMEM_SHARED((NP,), jnp.float32),
            pltpu.MemorySpace.VMEM_SHARED((CPX,), jnp.float32),
            pltpu.MemorySpace.VMEM_SHARED((16, 1, CPX), jnp.int32),
            pltpu.SemaphoreType.DMA,
            pltpu.SemaphoreType.DMA,
            [pltpu.SemaphoreType.DMA] * 2,
            [pltpu.SemaphoreType.DMA] * 2,
        ],
    )
    def k(ntc_h, bat_h, ed_h, w_h, psrc_h, pdst_h, deg1_h, deg2_h, cb_h,
          ed_v, w_v, ps_v, pd_v, ntc_v, bat_v, cb_v, zv,
          deg1_sh, deg2_sh, cb_st, sem, sem2, wsa, wsb):
        cid = lax.axis_index("c")
        sid = lax.axis_index("s")
        wid = sid * 2 + cid

        def zb(j, carry):
            zv[pl.ds(j * 16, 16)] = _z16f()
            return carry
        lax.fori_loop(0, 40, zb, 0)
        pltpu.sync_copy(zv, deg1_sh.at[pl.ds(sid * 640, 640)])
        pltpu.sync_copy(zv.at[pl.ds(0, 128)], deg2_sh.at[pl.ds(sid * 128, 128)])

        def zcb(j, carry):
            cb_v[pl.ds(j * 16, 16)] = _z16i()
            return carry
        lax.fori_loop(0, CPX // 16, zcb, 0)

        nbase = wid * (NP // 32)
        pltpu.sync_copy(ntc_h.at[pl.ds(nbase, 320)], ntc_v.at[pl.ds(0, 320)])
        pltpu.sync_copy(bat_h.at[pl.ds(nbase, 320)], bat_v.at[pl.ds(0, 320)])
        lanes = lax.iota(jnp.int32, 16)

        def nb(g, carry):
            c16 = ntc_v[pl.ds(g * 16, 16)]
            b16 = bat_v[pl.ds(g * 16, 16)]
            for kk in range(16):
                c = c16[kk]
                b = b16[kk]
                ca = (c >> 4) << 4
                lane = c - ca
                cur = cb_v[pl.ds(ca, 16)]
                cb_v[pl.ds(ca, 16)] = jnp.where(
                    lanes == lane, jnp.maximum(cur, b), cur)
            return carry
        lax.fori_loop(0, 20, nb, 0)
        pltpu.sync_copy(cb_v, cb_st.at[sid, 0])
        plsc.subcore_barrier()

        cbase = wid * NCH

        def wait_w(slot, ch):
            pltpu.make_async_copy(
                ps_v.at[slot], psrc_h.at[pl.ds((cbase + ch) * EK, EK)],
                wsa[slot]).wait()
            pltpu.make_async_copy(
                pd_v.at[slot], pdst_h.at[pl.ds((cbase + ch) * EK, EK)],
                wsb[slot]).wait()

        def eb(ch, slot):
            off = (cbase + ch) * EK
            pltpu.sync_copy(ed_h.at[cbase + ch], ed_v)
            pltpu.sync_copy(w_h.at[pl.ds(off, EK)], w_v)
            cp1 = pltpu.async_copy(ntc_h.at[ed_v.at[0]], ps_v.at[slot], sem)
            cp2 = pltpu.async_copy(ntc_h.at[ed_v.at[1]], pd_v.at[slot], sem2)
            cp1.wait()
            cp2.wait()
            pltpu.async_copy(ps_v.at[slot], psrc_h.at[pl.ds(off, EK)],
                             wsa[slot])
            pltpu.async_copy(pd_v.at[slot], pdst_h.at[pl.ds(off, EK)],
                             wsb[slot])
            pltpu.sync_copy(w_v, deg1_sh.at[ed_v.at[1]], add=True)
            pltpu.sync_copy(w_v, deg2_sh.at[pd_v.at[slot]], add=True)

        def ebp(pp, carry):
            for s_ in range(2):
                ch = 2 * pp + s_

                @pl.when(ch < NCH)
                def _():
                    @pl.when(ch >= 2)
                    def _():
                        wait_w(s_, ch - 2)
                    eb(ch, s_)
            return carry
        lax.fori_loop(0, (NCH + 1) // 2, ebp, 0)
        wait_w(0, NCH - 1)
        wait_w(1, NCH - 2)
        plsc.subcore_barrier()

        cslice = sid * 128
        pltpu.sync_copy(cb_st.at[0, 0, pl.ds(cslice, 128)],
                        cb_v.at[pl.ds(0, 128)])
        for t in range(1, 16):
            pltpu.sync_copy(
                cb_st.at[t, 0, pl.ds(cslice, 128)], cb_v.at[pl.ds(128, 128)])
            for j in range(0, 128, 16):
                cb_v[pl.ds(j, 16)] = jnp.maximum(
                    cb_v[pl.ds(j, 16)], cb_v[pl.ds(128 + j, 16)])
        pltpu.sync_copy(cb_v.at[pl.ds(0, 128)],
                        cb_h.at[cid, 0, pl.ds(cslice, 128)])
        pltpu.sync_copy(deg1_sh.at[pl.ds(sid * 640, 640)],
                        deg1_h.at[cid, 0, pl.ds(sid * 640, 640)])
        pltpu.sync_copy(deg2_sh.at[pl.ds(sid * 128, 128)],
                        deg2_h.at[cid, 0, pl.ds(sid * 128, 128)])

    return k(ntc, batch, ed, wp)


def _sc_agg(table, edata, wp, npad):
    RT = npad // 16

    @functools.partial(
        pl.kernel,
        out_type=jax.ShapeDtypeStruct((2, npad, H), jnp.float32),
        mesh=_mesh,
        scratch_types=[
            pltpu.VMEM((2, 2, EK), jnp.int32),
            pltpu.VMEM((2, EK), jnp.float32),
            pltpu.VMEM((2, EK, H), jnp.float32),
            pltpu.VMEM((16, H), jnp.float32),
            pltpu.MemorySpace.VMEM_SHARED((npad, H), jnp.float32),
            pltpu.SemaphoreType.DMA,
            pltpu.SemaphoreType.DMA,
        ],
    )
    def k(tab_h, ed_h, w_h, out_h, ed_v, w_v, rows_v, zbuf, acc_sh, sg0, sg1):
        cid = lax.axis_index("c")
        sid = lax.axis_index("s")
        wid = sid * 2 + cid
        for r in range(16):
            for j in range(0, H, 16):
                zbuf[r, pl.ds(j, 16)] = _z16f()

        def zb(z, carry):
            pltpu.sync_copy(zbuf, acc_sh.at[pl.ds(sid * RT + z * 16, 16)])
            return carry
        lax.fori_loop(0, RT // 16, zb, 0)
        plsc.subcore_barrier()

        cbase = wid * NCH
        sems = (sg0, sg1)

        def load_and_gather(ch, slot):
            pltpu.sync_copy(ed_h.at[cbase + ch], ed_v.at[slot])
            pltpu.sync_copy(
                w_h.at[pl.ds((cbase + ch) * EK, EK)], w_v.at[slot])
            pltpu.async_copy(tab_h.at[ed_v.at[slot, 0]], rows_v.at[slot],
                             sems[slot])

        def consume(slot):
            pltpu.make_async_copy(
                tab_h.at[ed_v.at[slot, 0]], rows_v.at[slot],
                sems[slot]).wait()

            def sc(g, carry2):
                w16 = w_v[slot, pl.ds(g * 16, 16)]
                for kk in range(16):
                    e = g * 16 + kk
                    wk = w16[kk]
                    for j in range(0, H, 16):
                        rows_v[slot, e, pl.ds(j, 16)] = (
                            rows_v[slot, e, pl.ds(j, 16)] * wk)
                return carry2
            lax.fori_loop(0, EK // 16, sc, 0, unroll=2)
            pltpu.sync_copy(rows_v.at[slot], acc_sh.at[ed_v.at[slot, 1]],
                            add=True)

        load_and_gather(0, 0)

        def body(p, carry):
            ch0 = 2 * p
            ch1 = ch0 + 1

            @pl.when(ch1 < NCH)
            def _():
                load_and_gather(ch1, 1)
            consume(0)

            @pl.when(ch0 + 2 < NCH)
            def _():
                load_and_gather(ch0 + 2, 0)

            @pl.when(ch1 < NCH)
            def _():
                consume(1)
            return carry
        lax.fori_loop(0, (NCH + 1) // 2, body, 0)
        plsc.subcore_barrier()
        pltpu.sync_copy(acc_sh.at[pl.ds(sid * RT, RT)],
                        out_h.at[cid, pl.ds(sid * RT, RT)])

    return k(table, edata, wp)


def _sc_pool1(h_fc, ntc, batch):
    RP = NP // 4
    HCH = 128
    CL = CPX * 16
    BL = B * 16

    @functools.partial(
        pl.kernel,
        out_type=[
            jax.ShapeDtypeStruct((2, 8, 1, CL), jnp.float32),
            jax.ShapeDtypeStruct((2, 8, 1, CL), jnp.float32),
            jax.ShapeDtypeStruct((2, 8, 1, BL), jnp.float32),
            jax.ShapeDtypeStruct((2, 8, 1, BL), jnp.float32),
        ],
        mesh=_mesh,
        scratch_types=[
            pltpu.VMEM((HCH, 16), jnp.float32),
            pltpu.VMEM((RP,), jnp.int32),
            pltpu.VMEM((RP,), jnp.int32),
            pltpu.VMEM((CL,), jnp.float32),
            pltpu.VMEM((CL,), jnp.float32),
            pltpu.VMEM((CL // 4,), jnp.float32),
            pltpu.VMEM((BL,), jnp.float32),
            pltpu.VMEM((BL,), jnp.float32),
            pltpu.MemorySpace.VMEM_SHARED((8, 1, CL), jnp.float32),
            pltpu.MemorySpace.VMEM_SHARED((8, 1, BL), jnp.float32),
            pltpu.MemorySpace.VMEM_SHARED((8, 1, BL), jnp.float32),
        ],
    )
    def k(h_h, ntc_h, bat_h, xadd_h, xmax_h, bsum_h, bmax_h,
          hbuf, ntc_v, bat_v, accs, accm, cmb, bs, bm, st_s, st_bs,
          st_bm):
        cid = lax.axis_index("c")
        sid = lax.axis_index("s")
        nr = sid // 8
        fc = sid % 8
        row0 = cid * (2 * RP) + nr * RP
        pltpu.sync_copy(ntc_h.at[pl.ds(row0, RP)], ntc_v)
        pltpu.sync_copy(bat_h.at[pl.ds(row0, RP)], bat_v)

        def zb(j, carry):
            accs[pl.ds(j * 16, 16)] = _z16f()
            accm[pl.ds(j * 16, 16)] = _z16f()
            return carry
        lax.fori_loop(0, CL // 16, zb, 0)
        for b in range(B):
            bs[pl.ds(b * 16, 16)] = _z16f()
            bm[pl.ds(b * 16, 16)] = _z16f()

        def chunk(ch, carry):
            pltpu.sync_copy(h_h.at[fc, pl.ds(row0 + ch * HCH, HCH)], hbuf)

            def body(g, carry2):
                c16 = ntc_v[pl.ds(ch * HCH + g * 16, 16)]
                b16 = bat_v[pl.ds(ch * HCH + g * 16, 16)]
                for kk in range(16):
                    c = c16[kk] * 16
                    b = b16[kk] * 16
                    row = hbuf[g * 16 + kk, :]
                    accs[pl.ds(c, 16)] = accs[pl.ds(c, 16)] + row
                    accm[pl.ds(c, 16)] = jnp.maximum(accm[pl.ds(c, 16)], row)
                    bs[pl.ds(b, 16)] = bs[pl.ds(b, 16)] + row
                    bm[pl.ds(b, 16)] = jnp.maximum(bm[pl.ds(b, 16)], row)
                return carry2
            lax.fori_loop(0, HCH // 16, body, 0)
            return carry
        lax.fori_loop(0, RP // HCH, chunk, 0)

        @pl.when(nr == 1)
        def _():
            pltpu.sync_copy(accs, st_s.at[fc, 0])
            pltpu.sync_copy(bs, st_bs.at[fc, 0])
            pltpu.sync_copy(bm, st_bm.at[fc, 0])
        plsc.subcore_barrier()

        QL = CL // 4

        @pl.when(nr == 0)
        def _():
            for q in range(4):
                pltpu.sync_copy(st_s.at[fc, 0, pl.ds(q * QL, QL)], cmb)

                def cb(j, carry):
                    accs[pl.ds(q * QL + j * 16, 16)] = (
                        accs[pl.ds(q * QL + j * 16, 16)]
                        + cmb[pl.ds(j * 16, 16)])
                    return carry
                lax.fori_loop(0, QL // 16, cb, 0)
        plsc.subcore_barrier()

        @pl.when(nr == 1)
        def _():
            pltpu.sync_copy(accm, st_s.at[fc, 0])
        plsc.subcore_barrier()

        @pl.when(nr == 0)
        def _():
            for q in range(4):
                pltpu.sync_copy(st_s.at[fc, 0, pl.ds(q * QL, QL)], cmb)

                def cb2(j, carry):
                    accm[pl.ds(q * QL + j * 16, 16)] = jnp.maximum(
                        accm[pl.ds(q * QL + j * 16, 16)],
                        cmb[pl.ds(j * 16, 16)])
                    return carry
                lax.fori_loop(0, QL // 16, cb2, 0)
            pltpu.sync_copy(st_bs.at[fc, 0], cmb.at[pl.ds(0, BL)])
            for b in range(B):
                bs[pl.ds(b * 16, 16)] = (
                    bs[pl.ds(b * 16, 16)] + cmb[pl.ds(b * 16, 16)])
            pltpu.sync_copy(st_bm.at[fc, 0], cmb.at[pl.ds(0, BL)])
            for b in range(B):
                bm[pl.ds(b * 16, 16)] = jnp.maximum(
                    bm[pl.ds(b * 16, 16)], cmb[pl.ds(b * 16, 16)])
            pltpu.sync_copy(accs, xadd_h.at[cid, fc, 0])
            pltpu.sync_copy(accm, xmax_h.at[cid, fc, 0])
            pltpu.sync_copy(bs, bsum_h.at[cid, fc, 0])
            pltpu.sync_copy(bm, bmax_h.at[cid, fc, 0])

    return k(h_fc, ntc, batch)


def _sc_pool2(h2_fc, cbp):
    RP = CPX // 4
    HCH = 128
    BL = B * 16

    @functools.partial(
        pl.kernel,
        out_type=[
            jax.ShapeDtypeStruct((2, 8, 1, BL), jnp.float32),
            jax.ShapeDtypeStruct((2, 8, 1, BL), jnp.float32),
        ],
        mesh=_mesh,
        scratch_types=[
            pltpu.VMEM((HCH, 16), jnp.float32),
            pltpu.VMEM((RP,), jnp.int32),
            pltpu.VMEM((RP,), jnp.int32),
            pltpu.VMEM((BL,), jnp.float32),
            pltpu.VMEM((BL,), jnp.float32),
            pltpu.VMEM((BL,), jnp.float32),
            pltpu.MemorySpace.VMEM_SHARED((8, 1, BL), jnp.float32),
            pltpu.MemorySpace.VMEM_SHARED((8, 1, BL), jnp.float32),
        ],
    )
    def k(h_h, cb_h, bsum_h, bmax_h, hbuf, s0_v, s1_v, bs, bm, cmb, st_bs,
          st_bm):
        cid = lax.axis_index("c")
        sid = lax.axis_index("s")
        nr = sid // 8
        fc = sid % 8
        row0 = cid * (2 * RP) + nr * RP
        pltpu.sync_copy(cb_h.at[0, 0, pl.ds(row0, RP)], s0_v)
        pltpu.sync_copy(cb_h.at[1, 0, pl.ds(row0, RP)], s1_v)

        def mseg(j, carry):
            m = jnp.maximum(s0_v[pl.ds(j * 16, 16)], s1_v[pl.ds(j * 16, 16)])
            s0_v[pl.ds(j * 16, 16)] = jnp.clip(m, 0, B - 1)
            return carry
        lax.fori_loop(0, RP // 16, mseg, 0)
        for b in range(B):
            bs[pl.ds(b * 16, 16)] = _z16f()
            bm[pl.ds(b * 16, 16)] = _z16f()

        def chunk(ch, carry):
            pltpu.sync_copy(h_h.at[fc, pl.ds(row0 + ch * HCH, HCH)], hbuf)

            def body(g, carry2):
                b16 = s0_v[pl.ds(ch * HCH + g * 16, 16)]
                for kk in range(16):
                    b = b16[kk] * 16
                    row = hbuf[g * 16 + kk, :]
                    bs[pl.ds(b, 16)] = bs[pl.ds(b, 16)] + row
                    bm[pl.ds(b, 16)] = jnp.maximum(bm[pl.ds(b, 16)], row)
                return carry2
            lax.fori_loop(0, HCH // 16, body, 0)
            return carry
        lax.fori_loop(0, RP // HCH, chunk, 0)

        @pl.when(nr == 1)
        def _():
            pltpu.sync_copy(bs, st_bs.at[fc, 0])
            pltpu.sync_copy(bm, st_bm.at[fc, 0])
        plsc.subcore_barrier()

        @pl.when(nr == 0)
        def _():
            pltpu.sync_copy(st_bs.at[fc, 0], cmb)
            for b in range(B):
                bs[pl.ds(b * 16, 16)] = (
                    bs[pl.ds(b * 16, 16)] + cmb[pl.ds(b * 16, 16)])
            pltpu.sync_copy(st_bm.at[fc, 0], cmb)
            for b in range(B):
                bm[pl.ds(b * 16, 16)] = jnp.maximum(
                    bm[pl.ds(b * 16, 16)], cmb[pl.ds(b * 16, 16)])
            pltpu.sync_copy(bs, bsum_h.at[cid, fc, 0])
            pltpu.sync_copy(bm, bmax_h.at[cid, fc, 0])

    return k(h2_fc, cbp)


def _dinv_of(degT_blk):
    d = degT_blk[:, 0] + degT_blk[:, 1] + 1.0
    return lax.rsqrt(jnp.clip(d, 1e-12, None))


def _tc_mm_scale(x, W, degT, npad):
    def body(x_ref, w_ref, deg_ref, h_ref, g_ref):
        h = jnp.dot(x_ref[...], w_ref[...], preferred_element_type=jnp.float32)
        dinv = _dinv_of(deg_ref[...])
        h_ref[...] = h
        g_ref[...] = dinv[:, None] * h

    grid = npad // RB
    return pl.pallas_call(
        body,
        grid=(grid,),
        in_specs=[
            pl.BlockSpec((RB, H), lambda i: (i, 0)),
            pl.BlockSpec((H, H), lambda i: (0, 0)),
            pl.BlockSpec((RB, 2), lambda i: (i, 0)),
        ],
        out_specs=[
            pl.BlockSpec((RB, H), lambda i: (i, 0)),
            pl.BlockSpec((RB, H), lambda i: (i, 0)),
        ],
        out_shape=[
            jax.ShapeDtypeStruct((npad, H), jnp.float32),
            jax.ShapeDtypeStruct((npad, H), jnp.float32),
        ],
    )(x, W, degT)


def _tc_gcn_epilogue(p0, p1, hprev, degT, bias, W2, npad, make_g):
    def body(p0_ref, p1_ref, h_ref, deg_ref, b_ref, w_ref,
             x1_ref, h1_ref, g1_ref):
        dinv = _dinv_of(deg_ref[...])
        agg = p0_ref[...] + p1_ref[...]
        x1 = jnp.maximum(
            dinv[:, None] * agg + (dinv * dinv)[:, None] * h_ref[...]
            + b_ref[...], 0.0)
        x1_ref[...] = x1
        h1 = jnp.dot(x1, w_ref[...], preferred_element_type=jnp.float32)
        h1_ref[...] = h1
        g1_ref[...] = dinv[:, None] * h1

    grid = npad // RB
    return pl.pallas_call(
        body,
        grid=(grid,),
        in_specs=[
            pl.BlockSpec((RB, H), lambda i: (i, 0)),
            pl.BlockSpec((RB, H), lambda i: (i, 0)),
            pl.BlockSpec((RB, H), lambda i: (i, 0)),
            pl.BlockSpec((RB, 2), lambda i: (i, 0)),
            pl.BlockSpec((1, H), lambda i: (0, 0)),
            pl.BlockSpec((H, H), lambda i: (0, 0)),
        ],
        out_specs=[
            pl.BlockSpec((RB, H), lambda i: (i, 0)),
            pl.BlockSpec((RB, H), lambda i: (i, 0)),
            pl.BlockSpec((RB, H), lambda i: (i, 0)),
        ],
        out_shape=[
            jax.ShapeDtypeStruct((npad, H), jnp.float32),
            jax.ShapeDtypeStruct((npad, H), jnp.float32),
            jax.ShapeDtypeStruct((npad, H), jnp.float32),
        ],
    )(p0, p1, hprev, degT, bias, W2)


def _tc_block_out(p0, p1, hprev, degT, bias, x1, Wla, Wlb, bl, npad, nvalid):
    def body(p0_ref, p1_ref, h_ref, deg_ref, b_ref, x1_ref, wa_ref, wb_ref,
             bl_ref, out_ref):
        i = pl.program_id(0)
        dinv = _dinv_of(deg_ref[...])
        agg = p0_ref[...] + p1_ref[...]
        x2 = jnp.maximum(
            dinv[:, None] * agg + (dinv * dinv)[:, None] * h_ref[...]
            + b_ref[...], 0.0)
        h = jnp.maximum(
            jnp.dot(x1_ref[...], wa_ref[...], preferred_element_type=jnp.float32)
            + jnp.dot(x2, wb_ref[...], preferred_element_type=jnp.float32)
            + bl_ref[...], 0.0)
        rows = i * RB + lax.broadcasted_iota(jnp.int32, (RB, 1), 0)
        out_ref[...] = jnp.where(rows < nvalid, h, 0.0)

    grid = npad // RB
    return pl.pallas_call(
        body,
        grid=(grid,),
        in_specs=[
            pl.BlockSpec((RB, H), lambda i: (i, 0)),
            pl.BlockSpec((RB, H), lambda i: (i, 0)),
            pl.BlockSpec((RB, H), lambda i: (i, 0)),
            pl.BlockSpec((RB, 2), lambda i: (i, 0)),
            pl.BlockSpec((1, H), lambda i: (0, 0)),
            pl.BlockSpec((RB, H), lambda i: (i, 0)),
            pl.BlockSpec((H, H), lambda i: (0, 0)),
            pl.BlockSpec((H, H), lambda i: (0, 0)),
            pl.BlockSpec((1, H), lambda i: (0, 0)),
        ],
        out_specs=pl.BlockSpec((RB, H), lambda i: (i, 0)),
        out_shape=jax.ShapeDtypeStruct((npad, H), jnp.float32),
    )(p0, p1, hprev, degT, bias, x1, Wla, Wlb, bl)


def _tc_cluster_in(xa0, xa1, xm0, xm1, degT, Ba, Bb):
    def body(a0_ref, a1_ref, m0_ref, m1_ref, deg_ref, ba_ref, bb_ref,
             h_ref, g_ref):
        dinv = _dinv_of(deg_ref[...])
        xadd = a0_ref[...] + a1_ref[...]
        xmax = jnp.maximum(m0_ref[...], m1_ref[...])
        h = (jnp.dot(xadd, ba_ref[...], preferred_element_type=jnp.float32)
             + jnp.dot(xmax, bb_ref[...], preferred_element_type=jnp.float32))
        h_ref[...] = h
        g_ref[...] = dinv[:, None] * h

    grid = CPX // RB
    return pl.pallas_call(
        body,
        grid=(grid,),
        in_specs=[
            pl.BlockSpec((RB, H), lambda i: (i, 0)),
            pl.BlockSpec((RB, H), lambda i: (i, 0)),
            pl.BlockSpec((RB, H), lambda i: (i, 0)),
            pl.BlockSpec((RB, H), lambda i: (i, 0)),
            pl.BlockSpec((RB, 2), lambda i: (i, 0)),
            pl.BlockSpec((H, H), lambda i: (0, 0)),
            pl.BlockSpec((H, H), lambda i: (0, 0)),
        ],
        out_specs=[
            pl.BlockSpec((RB, H), lambda i: (i, 0)),
            pl.BlockSpec((RB, H), lambda i: (i, 0)),
        ],
        out_shape=[
            jax.ShapeDtypeStruct((CPX, H), jnp.float32),
            jax.ShapeDtypeStruct((CPX, H), jnp.float32),
        ],
    )(xa0, xa1, xm0, xm1, degT, Ba, Bb)


def _tc_head(bs1p, bm1p, bs2p, bm2p, bn_g, bn_b, l1W, l1b, l2W, l2b):
    def body(s1_ref, m1_ref, s2_ref, m2_ref, g_ref, b_ref, w1_ref, b1_ref,
             w2_ref, b2_ref, out_ref):
        bs1 = s1_ref[0] + s1_ref[1]
        bm1 = jnp.maximum(m1_ref[0], m1_ref[1])
        bs2 = s2_ref[0] + s2_ref[1]
        bm2 = jnp.maximum(m2_ref[0], m2_ref[1])
        z = jnp.concatenate([bs1, bm1, bs2, bm2], axis=1)
        mean = jnp.mean(z, axis=0, keepdims=True)
        var = jnp.mean((z - mean) ** 2, axis=0, keepdims=True)
        z = (z - mean) / jnp.sqrt(var + 1e-5) * g_ref[...] + b_ref[...]
        hh = jnp.maximum(
            jnp.dot(z, w1_ref[...], preferred_element_type=jnp.float32)
            + b1_ref[...], 0.0)
        o = (jnp.dot(hh, w2_ref[...], preferred_element_type=jnp.float32)
             + b2_ref[...])
        o = o - jnp.max(o, axis=-1, keepdims=True)
        e = jnp.exp(o)
        out_ref[...] = e / jnp.sum(e, axis=-1, keepdims=True)

    return pl.pallas_call(
        body,
        out_shape=jax.ShapeDtypeStruct((B, NCLS), jnp.float32),
    )(bs1p, bm1p, bs2p, bm2p, bn_g[None, :], bn_b[None, :], l1W, l1b[None, :],
      l2W, l2b[None, :])


def _fc_layout(h, nvalid, npad):
    return jnp.transpose(h.reshape(npad, 8, 16), (1, 0, 2))


def kernel(x, edge_weight, cW1, cb1, cW2, cb2, cWl, cbl, bW1, bb1, bW2, bb2,
           bWl, bbl, bn_g, bn_b, l1W, l1b, l2W, l2b, edge_index,
           node_to_cluster, batch):
    src, dst = edge_index[0], edge_index[1]
    srcp = jnp.pad(src, (0, EP - E))
    dstp = jnp.pad(dst, (0, EP - E))
    wp = jnp.pad(edge_weight, (0, EP - E))
    ntcp = jnp.pad(node_to_cluster, (0, NP - N))
    batp = jnp.pad(batch, (0, NP - N))
    xp = jnp.pad(x, ((0, NP - N), (0, 0)))

    ed1 = jnp.concatenate(
        [srcp.reshape(EP // EK, 1, EK), dstp.reshape(EP // EK, 1, EK)], axis=1)
    psrc, pdst, deg1p, deg2p, cbp = _sc_pre(ntcp, batp, ed1, wp)
    ed2 = jnp.concatenate(
        [psrc.reshape(EP // EK, 1, EK), pdst.reshape(EP // EK, 1, EK)], axis=1)
    deg1T = jnp.transpose(deg1p.reshape(2, NP))
    deg2T = jnp.transpose(deg2p.reshape(2, CPX))

    h0, g0 = _tc_mm_scale(xp, cW1, deg1T, NP)
    a1 = _sc_agg(g0, ed1, wp, NP)
    x1, h1, g1 = _tc_gcn_epilogue(a1[0], a1[1], h0, deg1T, cb1[None, :], cW2,
                                  NP, True)
    a2 = _sc_agg(g1, ed1, wp, NP)
    h = _tc_block_out(a2[0], a2[1], h1, deg1T, cb2[None, :], x1,
                      cWl[:H], cWl[H:], cbl[None, :], NP, N)

    h_fc = _fc_layout(h, N, NP)
    xaddp, xmaxp, bs1p, bm1p = _sc_pool1(h_fc, ntcp, batp)

    def _cl(t):
        return jnp.transpose(
            t.reshape(2, 8, CPX, 16), (0, 2, 1, 3)).reshape(2, CPX, H)

    xa = _cl(xaddp)
    xm = _cl(xmaxp)

    hb0, g2 = _tc_cluster_in(xa[0], xa[1], xm[0], xm[1], deg2T,
                             bW1[:H], bW1[H:])
    a3 = _sc_agg(g2, ed2, wp, CPX)
    y1, h3, g3 = _tc_gcn_epilogue(a3[0], a3[1], hb0, deg2T, bb1[None, :], bW2,
                                  CPX, True)
    a4 = _sc_agg(g3, ed2, wp, CPX)
    h2 = _tc_block_out(a4[0], a4[1], h3, deg2T, bb2[None, :], y1,
                       bWl[:H], bWl[H:], bbl[None, :], CPX, C)

    h2_fc = _fc_layout(h2, C, CPX)
    bs2p, bm2p = _sc_pool2(h2_fc, cbp)

    def _bt(t):
        return jnp.transpose(t.reshape(2, 8, B, 16), (0, 2, 1, 3)).reshape(2, B, H)

    return _tc_head(_bt(bs1p), _bt(bm1p), _bt(bs2p), _bt(bm2p),
                    bn_g, bn_b, l1W, l1b, l2W, l2b)
